# baseline (speedup 1.0000x reference)
import jax
import jax.numpy as jnp
from jax import lax
from jax.experimental import pallas as pl
from jax.experimental.pallas import tpu as pltpu
from jax.experimental.pallas import tpu_sc as plsc

_B, _T, _D = 4, 2048, 128
_NW = 32
_ROWS = _B * _T // _NW
_CH = 64
_NCH = _ROWS // _CH


def _body(idx_hbm, tok_hbm, pos_hbm, out_hbm, idx_v, rows_v, pos_v,
          gsem, psem, osem):
    wid = lax.axis_index("s") * 2 + lax.axis_index("c")
    base = wid * _ROWS
    batch = lax.div(base, _T)
    pos_base = lax.rem(base, _T)

    pltpu.sync_copy(idx_hbm.at[batch, pl.ds(pos_base, _ROWS)], idx_v)
    gathers, poses = [], []
    for c in range(_NCH):
        lo = c * _CH
        gathers.append(
            pltpu.async_copy(tok_hbm.at[idx_v.at[pl.ds(lo, _CH)]],
                             rows_v.at[pl.ds(lo, _CH)], gsem.at[c]))
        poses.append(
            pltpu.async_copy(pos_hbm.at[pl.ds(pos_base + lo, _CH)],
                             pos_v.at[pl.ds(lo, _CH)], psem.at[c]))

    stores = []
    for c in range(_NCH):
        gathers[c].wait()
        poses[c].wait()
        lo = c * _CH

        @plsc.parallel_loop(lo, lo + _CH, unroll=4)
        def _add(r):
            for k in range(_D // 16):
                sl = pl.ds(k * 16, 16)
                plsc.addupdate(rows_v.at[r, sl], pos_v[r, sl])

        stores.append(
            pltpu.async_copy(rows_v.at[pl.ds(lo, _CH)],
                             out_hbm.at[pl.ds(base + lo, _CH)], osem.at[c]))
    for cp in stores:
        cp.wait()


def kernel(idx, token_table, pos_table):
    mesh = plsc.VectorSubcoreMesh(core_axis_name="c", subcore_axis_name="s")
    f = pl.kernel(
        _body,
        out_type=jax.ShapeDtypeStruct((_B * _T, _D), jnp.float32),
        mesh=mesh,
        scratch_types=[
            pltpu.VMEM((_ROWS,), jnp.int32),
            pltpu.VMEM((_ROWS, _D), jnp.float32),
            pltpu.VMEM((_ROWS, _D), jnp.float32),
            pltpu.SemaphoreType.DMA((_NCH,)),
            pltpu.SemaphoreType.DMA((_NCH,)),
            pltpu.SemaphoreType.DMA((_NCH,)),
        ],
    )
    out = f(idx, token_table, pos_table)
    return out.reshape(_B, _T, _D)


# speedup vs baseline: 1.0595x; 1.0595x over previous
import jax
import jax.numpy as jnp
from jax import lax
from jax.experimental import pallas as pl
from jax.experimental.pallas import tpu as pltpu
from jax.experimental.pallas import tpu_sc as plsc

_B, _T, _D = 4, 2048, 128
_NW = 32
_ROWS = _B * _T // _NW
_CH = 64
_NCH = _ROWS // _CH


def _body(idx_hbm, tok_hbm, pos_hbm, out_hbm, idx_v, pos_v,
          gsem, psem, osem):
    wid = lax.axis_index("s") * 2 + lax.axis_index("c")
    base = wid * _ROWS
    batch = lax.div(base, _T)
    pos_base = lax.rem(base, _T)

    pltpu.sync_copy(idx_hbm.at[batch, pl.ds(pos_base, _ROWS)], idx_v)
    poses = [
        pltpu.async_copy(pos_hbm.at[pl.ds(pos_base + c * _CH, _CH)],
                         pos_v.at[pl.ds(c * _CH, _CH)], psem.at[c])
        for c in range(_NCH)
    ]
    gathers = []
    for c in range(_NCH):
        poses[c].wait()
        lo = c * _CH
        gathers.append(
            pltpu.async_copy(tok_hbm.at[idx_v.at[pl.ds(lo, _CH)]],
                             pos_v.at[pl.ds(lo, _CH)], gsem.at[c], add=True))

    stores = []
    for c in range(_NCH):
        gathers[c].wait()
        lo = c * _CH
        stores.append(
            pltpu.async_copy(pos_v.at[pl.ds(lo, _CH)],
                             out_hbm.at[pl.ds(base + lo, _CH)], osem.at[c]))
    for cp in stores:
        cp.wait()


def kernel(idx, token_table, pos_table):
    mesh = plsc.VectorSubcoreMesh(core_axis_name="c", subcore_axis_name="s")
    f = pl.kernel(
        _body,
        out_type=jax.ShapeDtypeStruct((_B * _T, _D), jnp.float32),
        mesh=mesh,
        scratch_types=[
            pltpu.VMEM((_ROWS,), jnp.int32),
            pltpu.VMEM((_ROWS, _D), jnp.float32),
            pltpu.SemaphoreType.DMA((_NCH,)),
            pltpu.SemaphoreType.DMA((_NCH,)),
            pltpu.SemaphoreType.DMA((_NCH,)),
        ],
    )
    out = f(idx, token_table, pos_table)
    return out.reshape(_B, _T, _D)


# speedup vs baseline: 1.0604x; 1.0009x over previous
import jax
import jax.numpy as jnp
from jax import lax
from jax.experimental import pallas as pl
from jax.experimental.pallas import tpu as pltpu
from jax.experimental.pallas import tpu_sc as plsc

_B, _T, _D = 4, 2048, 128
_NW = 32
_ROWS = _B * _T // _NW
_CH = 32
_NCH = _ROWS // _CH


def _body(idx_hbm, tok_hbm, pos_hbm, out_hbm, idx_v, pos_v,
          gsem, psem, osem):
    wid = lax.axis_index("s") * 2 + lax.axis_index("c")
    base = wid * _ROWS
    batch = lax.div(base, _T)
    pos_base = lax.rem(base, _T)

    pltpu.sync_copy(idx_hbm.at[batch, pl.ds(pos_base, _ROWS)], idx_v)
    poses = [
        pltpu.async_copy(pos_hbm.at[pl.ds(pos_base + c * _CH, _CH)],
                         pos_v.at[pl.ds(c * _CH, _CH)], psem.at[c])
        for c in range(_NCH)
    ]
    gathers = []
    for c in range(_NCH):
        poses[c].wait()
        lo = c * _CH
        gathers.append(
            pltpu.async_copy(tok_hbm.at[idx_v.at[pl.ds(lo, _CH)]],
                             pos_v.at[pl.ds(lo, _CH)], gsem.at[c], add=True))

    stores = []
    for c in range(_NCH):
        gathers[c].wait()
        lo = c * _CH
        stores.append(
            pltpu.async_copy(pos_v.at[pl.ds(lo, _CH)],
                             out_hbm.at[pl.ds(base + lo, _CH)], osem.at[c]))
    for cp in stores:
        cp.wait()


def kernel(idx, token_table, pos_table):
    mesh = plsc.VectorSubcoreMesh(core_axis_name="c", subcore_axis_name="s")
    f = pl.kernel(
        _body,
        out_type=jax.ShapeDtypeStruct((_B * _T, _D), jnp.float32),
        mesh=mesh,
        scratch_types=[
            pltpu.VMEM((_ROWS,), jnp.int32),
            pltpu.VMEM((_ROWS, _D), jnp.float32),
            pltpu.SemaphoreType.DMA((_NCH,)),
            pltpu.SemaphoreType.DMA((_NCH,)),
            pltpu.SemaphoreType.DMA((_NCH,)),
        ],
    )
    out = f(idx, token_table, pos_table)
    return out.reshape(_B, _T, _D)
